# Initial kernel scaffold; baseline (speedup 1.0000x reference)
import jax
import jax.numpy as jnp
from jax.experimental import pallas as pl


def kernel(input_pos_tensors, table):
    raise NotImplementedError("write your pallas kernel here")


# speedup vs baseline: 3.0981x; 3.0981x over previous
import functools

import jax
import jax.numpy as jnp
from jax.experimental import pallas as pl
from jax.experimental.pallas import tpu as pltpu
from jax.experimental.pallas import tpu_sc as plsc

_W = 256


def _gather_rows(idx_flat, table, n):
    d = table.shape[1]
    mesh = plsc.VectorSubcoreMesh(core_axis_name="c", subcore_axis_name="s")

    @functools.partial(
        pl.kernel,
        out_type=jax.ShapeDtypeStruct((n, d), table.dtype),
        mesh=mesh,
    )
    def k(table_hbm, idx_hbm, out_hbm):
        def body(idx_vmem, out_vmem):
            pltpu.sync_copy(table_hbm.at[idx_vmem.at[0]], out_vmem)

        pltpu.emit_pipeline(
            body,
            grid=(n // _W,),
            in_specs=[pl.BlockSpec((1, _W), index_map=lambda i: (0, i))],
            out_specs=[pl.BlockSpec((_W, d), index_map=lambda i: (i, 0))],
            core_axis_name=("c", "s"),
            dimension_semantics=(pltpu.PARALLEL,),
        )(idx_hbm, out_hbm)

    return k(table, idx_flat)


def kernel(input_pos_tensors, table):
    b, s = input_pos_tensors.shape
    n = b * s
    idx_flat = input_pos_tensors.reshape(1, n)
    out = _gather_rows(idx_flat, table, n)
    return out.reshape(b, s, table.shape[1])


# speedup vs baseline: 3.1404x; 1.0137x over previous
import functools

import jax
import jax.numpy as jnp
from jax.experimental import pallas as pl
from jax.experimental.pallas import tpu as pltpu
from jax.experimental.pallas import tpu_sc as plsc

_NC = 2
_NS = 16
_NW = _NC * _NS
_C = 400


def _gather_rows(idx_grouped, table, n):
    d = table.shape[1]
    nw, per_w = idx_grouped.shape
    c = _C
    g_chunks = per_w // c
    mesh = plsc.VectorSubcoreMesh(core_axis_name="c", subcore_axis_name="s")

    @functools.partial(
        pl.kernel,
        out_type=jax.ShapeDtypeStruct((n, d), table.dtype),
        mesh=mesh,
        scratch_types=[
            pltpu.VMEM((per_w,), jnp.int32),
            pltpu.VMEM((2, c, d), table.dtype),
            pltpu.SemaphoreType.DMA,
            pltpu.SemaphoreType.DMA,
            pltpu.SemaphoreType.DMA,
            pltpu.SemaphoreType.DMA,
        ],
    )
    def k(table_hbm, idx_hbm, out_hbm, idx_v, rows_v, gsem0, gsem1, wsem0, wsem1):
        wid = jax.lax.axis_index("s") * _NC + jax.lax.axis_index("c")
        base = wid * (g_chunks * c)
        pltpu.sync_copy(idx_hbm.at[wid], idx_v)

        gsems = (gsem0, gsem1)
        wsems = (wsem0, wsem1)

        def start_gather(chunk, buf):
            pltpu.async_copy(
                table_hbm.at[idx_v.at[pl.ds(chunk * c, c)]],
                rows_v.at[buf],
                gsems[buf],
            )

        def finish_chunk(chunk, buf):
            pltpu.make_async_copy(
                table_hbm.at[idx_v.at[pl.ds(0, c)]], rows_v.at[buf], gsems[buf]
            ).wait()
            pltpu.async_copy(
                rows_v.at[buf],
                out_hbm.at[pl.ds(base + chunk * c, c)],
                wsems[buf],
            )

        def wait_write(buf):
            pltpu.make_async_copy(
                rows_v.at[buf],
                out_hbm.at[pl.ds(base, c)],
                wsems[buf],
            ).wait()

        start_gather(0, 0)
        start_gather(1, 1)
        finish_chunk(0, 0)

        @pl.loop(2, g_chunks, step=2)
        def _(chunk0):
            for buf in (0, 1):
                chunk = chunk0 + buf
                other = 1 - buf
                pltpu.make_async_copy(
                    table_hbm.at[idx_v.at[pl.ds(0, c)]],
                    rows_v.at[other],
                    gsems[other],
                ).wait()
                pltpu.async_copy(
                    rows_v.at[other],
                    out_hbm.at[pl.ds(base + (chunk - 1) * c, c)],
                    wsems[other],
                )
                wait_write(buf)
                start_gather(chunk, buf)

        last = g_chunks - 1
        finish_chunk(last, last % 2)
        wait_write(0)
        wait_write(1)

    return k(table, idx_grouped)


def kernel(input_pos_tensors, table):
    b, s = input_pos_tensors.shape
    n = b * s
    idx_grouped = input_pos_tensors.reshape(_NW, n // _NW)
    out = _gather_rows(idx_grouped, table, n)
    return out.reshape(b, s, table.shape[1])


# speedup vs baseline: 5.3708x; 1.7102x over previous
import functools

import jax
import jax.numpy as jnp
from jax.experimental import pallas as pl
from jax.experimental.pallas import tpu as pltpu
from jax.experimental.pallas import tpu_sc as plsc

_NC = 2
_NS = 16
_NW = _NC * _NS
_BPC = 8


def _gather_rows(idx_grouped, table, b, s):
    d = table.shape[1]
    nw, per_w = idx_grouped.shape
    c = _BPC * s
    g_chunks = per_w // c
    batches_w = per_w // s
    mesh = plsc.VectorSubcoreMesh(core_axis_name="c", subcore_axis_name="s")

    @functools.partial(
        pl.kernel,
        out_type=jax.ShapeDtypeStruct((b, s, d), table.dtype),
        mesh=mesh,
        scratch_types=[
            pltpu.VMEM((per_w,), jnp.int32),
            pltpu.VMEM((2, c, d), table.dtype),
            pltpu.SemaphoreType.DMA,
            pltpu.SemaphoreType.DMA,
            pltpu.SemaphoreType.DMA,
            pltpu.SemaphoreType.DMA,
        ],
    )
    def k(table_hbm, idx_hbm, out_hbm, idx_v, rows_v, gsem0, gsem1, wsem0, wsem1):
        wid = jax.lax.axis_index("s") * _NC + jax.lax.axis_index("c")
        batch_base = wid * batches_w
        pltpu.sync_copy(idx_hbm.at[wid], idx_v)

        gsems = (gsem0, gsem1)
        wsems = (wsem0, wsem1)

        def start_gather(chunk, buf):
            pltpu.async_copy(
                table_hbm.at[idx_v.at[pl.ds(chunk * c, c)]],
                rows_v.at[buf],
                gsems[buf],
            )

        def finish_chunk(chunk, buf):
            pltpu.make_async_copy(
                table_hbm.at[idx_v.at[pl.ds(0, c)]], rows_v.at[buf], gsems[buf]
            ).wait()
            for i in range(_BPC):
                pltpu.async_copy(
                    rows_v.at[buf].at[pl.ds(i * s, s)],
                    out_hbm.at[batch_base + chunk * _BPC + i],
                    wsems[buf],
                )

        def wait_write(buf):
            for _ in range(_BPC):
                pltpu.make_async_copy(
                    rows_v.at[buf].at[pl.ds(0, s)],
                    out_hbm.at[0],
                    wsems[buf],
                ).wait()

        start_gather(0, 0)
        start_gather(1, 1)
        finish_chunk(0, 0)

        @pl.loop(2, g_chunks, step=2)
        def _(chunk0):
            for buf in (0, 1):
                chunk = chunk0 + buf
                other = 1 - buf
                finish_chunk(chunk - 1, other)
                wait_write(buf)
                start_gather(chunk, buf)

        last = g_chunks - 1
        finish_chunk(last, last % 2)
        wait_write(0)
        wait_write(1)

    return k(table, idx_grouped)


def kernel(input_pos_tensors, table):
    b, s = input_pos_tensors.shape
    n = b * s
    idx_grouped = input_pos_tensors.reshape(_NW, n // _NW)
    return _gather_rows(idx_grouped, table, b, s)


# speedup vs baseline: 8.9171x; 1.6603x over previous
import functools

import jax
import jax.numpy as jnp
from jax.experimental import pallas as pl
from jax.experimental.pallas import tpu as pltpu
from jax.experimental.pallas import tpu_sc as plsc

_NC = 2
_NS = 16
_NW = _NC * _NS
_C = 400


def _gather_rows(idx_grouped, table, n):
    d = table.shape[1]
    nw, per_w = idx_grouped.shape
    c = _C
    g_chunks = per_w // c
    mesh = plsc.VectorSubcoreMesh(core_axis_name="c", subcore_axis_name="s")

    @functools.partial(
        pl.kernel,
        out_type=jax.ShapeDtypeStruct((n, d), table.dtype),
        mesh=mesh,
        scratch_types=[
            pltpu.VMEM((per_w,), jnp.int32),
            pltpu.VMEM((2, c, d), table.dtype),
            pltpu.SemaphoreType.DMA,
            pltpu.SemaphoreType.DMA,
            pltpu.SemaphoreType.DMA,
            pltpu.SemaphoreType.DMA,
        ],
    )
    def k(table_hbm, idx_hbm, out_hbm, idx_v, rows_v, gsem0, gsem1, wsem0, wsem1):
        wid = jax.lax.axis_index("s") * _NC + jax.lax.axis_index("c")
        base = wid * per_w
        pltpu.sync_copy(idx_hbm.at[wid], idx_v)

        gsems = (gsem0, gsem1)
        wsems = (wsem0, wsem1)

        def start_gather(chunk, buf):
            pltpu.async_copy(
                table_hbm.at[idx_v.at[pl.ds(chunk * c, c)]],
                rows_v.at[buf],
                gsems[buf],
            )

        def finish_chunk(chunk, buf):
            pltpu.make_async_copy(
                table_hbm.at[idx_v.at[pl.ds(0, c)]], rows_v.at[buf], gsems[buf]
            ).wait()
            pltpu.async_copy(
                rows_v.at[buf],
                out_hbm.at[pl.ds(base + chunk * c, c)],
                wsems[buf],
            )

        def wait_write(buf):
            pltpu.make_async_copy(
                rows_v.at[buf],
                out_hbm.at[pl.ds(base, c)],
                wsems[buf],
            ).wait()

        start_gather(0, 0)
        start_gather(1, 1)
        finish_chunk(0, 0)

        @pl.loop(2, g_chunks, step=2)
        def _(chunk0):
            for buf in (0, 1):
                chunk = chunk0 + buf
                other = 1 - buf
                finish_chunk(chunk - 1, other)
                wait_write(buf)
                start_gather(chunk, buf)

        last = g_chunks - 1
        finish_chunk(last, last % 2)
        wait_write(0)
        wait_write(1)

    return k(table, idx_grouped)


def kernel(input_pos_tensors, table):
    b, s = input_pos_tensors.shape
    n = b * s
    d = table.shape[1]
    idx_grouped = input_pos_tensors.T.reshape(_NW, n // _NW)
    out = _gather_rows(idx_grouped, table, n)
    return jnp.swapaxes(out.reshape(s, b, d), 0, 1)
